# baseline (speedup 1.0000x reference)
import functools

import jax
import jax.numpy as jnp
from jax import lax
from jax.experimental import pallas as pl
from jax.experimental.pallas import tpu as pltpu
from jax.experimental.pallas import tpu_sc as plsc

N = 10000
E = 320000
D = 128
H = 128
EPS = 1e-5

NC = 2
NS = 16
NW = NC * NS
EPW = E // NW

_mesh = plsc.VectorSubcoreMesh(
    core_axis_name="c", subcore_axis_name="s", num_cores=NC, num_subcores=NS)


def _deg_body(dst_hbm, deg_hbm, dst_v, hist_v, sem):
    c = lax.axis_index("c")
    s = lax.axis_index("s")
    wid = c * jnp.int32(NS) + s

    zero16 = jnp.zeros((16,), jnp.float32)

    def zb(i, carry):
        hist_v[pl.ds(i * jnp.int32(16), 16)] = zero16
        return carry

    lax.fori_loop(jnp.int32(0), jnp.int32(N // 16), zb, jnp.int32(0))

    pltpu.sync_copy(dst_hbm.at[wid], dst_v)
    ones16 = jnp.ones((16,), jnp.float32)

    def body(i, carry):
        idx = dst_v[pl.ds(i * jnp.int32(16), 16)]
        plsc.addupdate_scatter(hist_v, [idx], ones16)
        return carry

    lax.fori_loop(jnp.int32(0), jnp.int32(EPW // 16), body, jnp.int32(0))

    pltpu.sync_copy(hist_v, deg_hbm.at[wid])


_deg_call = functools.partial(
    pl.kernel,
    out_type=jax.ShapeDtypeStruct((NW, N), jnp.float32),
    mesh=_mesh,
    compiler_params=pltpu.CompilerParams(needs_layout_passes=False),
    scratch_types=[
        pltpu.VMEM((EPW,), jnp.int32),
        pltpu.VMEM((N,), jnp.float32),
        pltpu.SemaphoreType.DMA,
    ],
)


def _y_body(x_ref, w_ref, degt_ref, y_ref):
    deg = jnp.sum(degt_ref[:, :], axis=1, keepdims=True) + 1.0
    dinv = lax.rsqrt(deg)
    xw = jnp.dot(x_ref[:, :], w_ref[:, :], preferred_element_type=jnp.float32,
                 precision=lax.Precision.HIGHEST)
    y_ref[:, :] = xw * dinv


RB = 2000


def _y_call(x, w1, degt):
    grid = N // RB
    return pl.pallas_call(
        _y_body,
        grid=(grid,),
        in_specs=[
            pl.BlockSpec((RB, D), lambda i: (i, jnp.int32(0))),
            pl.BlockSpec((D, H), lambda i: (jnp.int32(0), jnp.int32(0))),
            pl.BlockSpec((RB, NW), lambda i: (i, jnp.int32(0))),
        ],
        out_specs=pl.BlockSpec((RB, H), lambda i: (i, jnp.int32(0))),
        out_shape=jax.ShapeDtypeStruct((N, H), jnp.float32),
    )(x, w1, degt)


EPT = 10240
SCH = 64
NCH2 = EPT // SCH
GRP = 8
NG = NCH2 // GRP
NB = 4
NP = 10112
RPS = NP // NS
DPAD = 10016


def _agg_body(y_hbm, src_hbm, dst_hbm, agg_hbm, srcb, dstb, rows, shared,
              g0, g1, g2, g3, t0, t1, t2, t3, dsem):
    c = lax.axis_index("c")
    s = lax.axis_index("s")
    wid = c * jnp.int32(NS) + s
    gsems = (g0, g1, g2, g3)
    ssems = (t0, t1, t2, t3)

    zero16 = jnp.zeros((16,), jnp.float32)

    def zb(i, carry):
        for f in range(H // 16):
            rows[jnp.int32(0), i, pl.ds(f * 16, 16)] = zero16
        return carry

    lax.fori_loop(jnp.int32(0), jnp.int32(SCH), zb, jnp.int32(0))
    for k in range(9):
        pltpu.sync_copy(
            rows.at[jnp.int32(0)],
            shared.at[pl.ds(s * jnp.int32(RPS) + jnp.int32(k * SCH), SCH)])
    pltpu.sync_copy(
        rows.at[jnp.int32(0), pl.ds(0, RPS - 9 * SCH)],
        shared.at[pl.ds(s * jnp.int32(RPS) + jnp.int32(9 * SCH),
                        RPS - 9 * SCH)])

    pltpu.sync_copy(src_hbm.at[wid], srcb)
    plsc.subcore_barrier()

    pltpu.async_copy(dst_hbm.at[wid, jnp.int32(0)], dstb.at[jnp.int32(0)],
                     dsem)
    for b in range(NB - 1):
        pltpu.async_copy(
            y_hbm.at[srcb.at[pl.ds(jnp.int32(b * SCH), SCH)]],
            rows.at[jnp.int32(b)], gsems[b])

    def grp(g, carry):
        p = g % jnp.int32(2)
        pltpu.make_async_copy(dst_hbm.at[wid, g], dstb.at[p], dsem).wait()
        for k in range(GRP):
            b = k % NB
            j = g * jnp.int32(GRP) + jnp.int32(k)
            pltpu.make_async_copy(
                y_hbm.at[srcb.at[pl.ds(j * jnp.int32(SCH), SCH)]],
                rows.at[jnp.int32(b)], gsems[b]).wait()
            pltpu.async_copy(rows.at[jnp.int32(b)],
                             shared.at[dstb.at[p, jnp.int32(k)]], ssems[b],
                             add=True)
            bb = (k + NB - 1) % NB

            @pl.when(j + jnp.int32(NB - 1) < jnp.int32(NCH2))
            def _():
                @pl.when(j >= jnp.int32(1))
                def _():
                    pltpu.make_async_copy(
                        rows.at[jnp.int32(bb)],
                        shared.at[dstb.at[p, jnp.int32(k)]],
                        ssems[bb]).wait()

                pltpu.async_copy(
                    y_hbm.at[srcb.at[pl.ds(
                        (j + jnp.int32(NB - 1)) * jnp.int32(SCH), SCH)]],
                    rows.at[jnp.int32(bb)], gsems[bb])

            if k == 0:
                @pl.when(g + jnp.int32(1) < jnp.int32(NG))
                def _():
                    pltpu.async_copy(dst_hbm.at[wid, g + jnp.int32(1)],
                                     dstb.at[jnp.int32(1) - p], dsem)
        return carry

    lax.fori_loop(jnp.int32(0), jnp.int32(NG), grp, jnp.int32(0))
    for b in range(NB):
        pltpu.make_async_copy(
            rows.at[jnp.int32(b)],
            shared.at[dstb.at[jnp.int32(1), jnp.int32(NB + b)]],
            ssems[b]).wait()
    plsc.subcore_barrier()

    base = s * jnp.int32(RPS)
    pltpu.sync_copy(shared.at[pl.ds(base, RPS)],
                    agg_hbm.at[c, pl.ds(base, RPS)])


_agg_call = functools.partial(
    pl.kernel,
    out_type=jax.ShapeDtypeStruct((NC, NP, H), jnp.float32),
    mesh=_mesh,
    scratch_types=[
        pltpu.VMEM((EPT,), jnp.int32),
        pltpu.VMEM((2, GRP, SCH), jnp.int32),
        pltpu.VMEM((NB, SCH, H), jnp.float32),
        pltpu.VMEM_SHARED((NP, H), jnp.float32),
        pltpu.SemaphoreType.DMA,
        pltpu.SemaphoreType.DMA,
        pltpu.SemaphoreType.DMA,
        pltpu.SemaphoreType.DMA,
        pltpu.SemaphoreType.DMA,
        pltpu.SemaphoreType.DMA,
        pltpu.SemaphoreType.DMA,
        pltpu.SemaphoreType.DMA,
        pltpu.SemaphoreType.DMA,
    ],
)


def _h_body(agg_ref, y_ref, degt_ref, b1_ref, h_ref, st_ref):
    deg = jnp.sum(degt_ref[:, :], axis=1, keepdims=True) + 1.0
    dinv = lax.rsqrt(deg)
    tot = (agg_ref[0, :, :] + agg_ref[1, :, :] + y_ref[:, :]) * dinv
    h = jnp.maximum(tot + b1_ref[:, :], 0.0)
    h_ref[:, :] = h

    @pl.when(pl.program_id(0) == 0)
    def _():
        st_ref[:, :] = jnp.zeros_like(st_ref)

    st_ref[0:1, :] += jnp.sum(h, axis=0, keepdims=True)
    st_ref[1:2, :] += jnp.sum(h * h, axis=0, keepdims=True)


def _h_call(agg, y, degt, b1r):
    grid = N // RB
    return pl.pallas_call(
        _h_body,
        grid=(grid,),
        in_specs=[
            pl.BlockSpec((NC, RB, H), lambda i: (jnp.int32(0), i, jnp.int32(0))),
            pl.BlockSpec((RB, H), lambda i: (i, jnp.int32(0))),
            pl.BlockSpec((RB, NW), lambda i: (i, jnp.int32(0))),
            pl.BlockSpec((1, H), lambda i: (jnp.int32(0), jnp.int32(0))),
        ],
        out_specs=[
            pl.BlockSpec((RB, H), lambda i: (i, jnp.int32(0))),
            pl.BlockSpec((2, H), lambda i: (jnp.int32(0), jnp.int32(0))),
        ],
        out_shape=[
            jax.ShapeDtypeStruct((N, H), jnp.float32),
            jax.ShapeDtypeStruct((2, H), jnp.float32),
        ],
    )(agg, y, degt, b1r)


def _pq_body(h_ref, st_ref, g_ref, be_ref, w_ref, bfc_ref, pq_ref):
    inv_n = 1.0 / N
    mean = st_ref[0:1, :] * inv_n
    var = st_ref[1:2, :] * inv_n - mean * mean
    scale = g_ref[:, :] * lax.rsqrt(var + EPS)
    shift = be_ref[:, :] - mean * scale
    h2 = jnp.maximum(h_ref[:, :] * scale + shift, 0.0)
    pq_ref[:, :] = (jnp.dot(h2, w_ref[:, :],
                            preferred_element_type=jnp.float32,
                            precision=lax.Precision.HIGHEST)
                    + bfc_ref[:, :])


def _pq_call(h, st, gr, br, wcat, bfc):
    grid = N // RB
    return pl.pallas_call(
        _pq_body,
        grid=(grid,),
        in_specs=[
            pl.BlockSpec((RB, H), lambda i: (i, jnp.int32(0))),
            pl.BlockSpec((2, H), lambda i: (jnp.int32(0), jnp.int32(0))),
            pl.BlockSpec((1, H), lambda i: (jnp.int32(0), jnp.int32(0))),
            pl.BlockSpec((1, H), lambda i: (jnp.int32(0), jnp.int32(0))),
            pl.BlockSpec((H, 4), lambda i: (jnp.int32(0), jnp.int32(0))),
            pl.BlockSpec((1, 4), lambda i: (jnp.int32(0), jnp.int32(0))),
        ],
        out_specs=pl.BlockSpec((RB, 4), lambda i: (i, jnp.int32(0))),
        out_shape=jax.ShapeDtypeStruct((N, 4), jnp.float32),
    )(h, st, gr, br, wcat, bfc)


def _score_body(pq_hbm, src_hbm, dst_hbm, out_hbm, pqv, srcv, dstv, outv, sem):
    c = lax.axis_index("c")
    s = lax.axis_index("s")
    wid = c * jnp.int32(NS) + s

    pltpu.sync_copy(pq_hbm, pqv)
    pltpu.sync_copy(src_hbm.at[wid], srcv)
    pltpu.sync_copy(dst_hbm.at[wid], dstv)

    lane = lax.iota(jnp.int32, 16)
    expmask = jnp.int32(0x7F800000)
    magmask = jnp.int32(0x7FFFFFFF)
    signmask = jnp.int32(-2147483648)
    ebias = jnp.int32(896 << 20)
    zero16i = jnp.zeros((16,), jnp.int32)

    def f64bits(v):
        b = plsc.bitcast(v, jnp.int32)
        nz = (b & expmask) != zero16i
        hi = (b & signmask) | (((b & magmask) >> jnp.int32(3)) + ebias)
        lo = (b & jnp.int32(7)) << jnp.int32(29)
        return jnp.where(nz, lo, zero16i), jnp.where(nz, hi, zero16i)

    def body(i, carry):
        sv = srcv[pl.ds(i * jnp.int32(16), 16)] * jnp.int32(4)
        dv = dstv[pl.ds(i * jnp.int32(16), 16)] * jnp.int32(4)
        p0 = plsc.load_gather(pqv, [sv])
        p1 = plsc.load_gather(pqv, [sv + jnp.int32(1)])
        q0 = plsc.load_gather(pqv, [dv + jnp.int32(2)])
        q1 = plsc.load_gather(pqv, [dv + jnp.int32(3)])
        lo0, hi0 = f64bits(p0 + q0)
        lo1, hi1 = f64bits(p1 + q1)
        base = i * jnp.int32(16)
        outv[0, pl.ds(base, 16)] = lo0
        outv[1, pl.ds(base, 16)] = lo1
        outv[2, pl.ds(base, 16)] = hi0
        outv[3, pl.ds(base, 16)] = hi1
        return carry

    lax.fori_loop(jnp.int32(0), jnp.int32(EPW // 16), body, jnp.int32(0))
    for p in range(4):
        pltpu.sync_copy(outv.at[jnp.int32(p)],
                        out_hbm.at[jnp.int32(p * NW) + wid])


_score_call = functools.partial(
    pl.kernel,
    out_type=jax.ShapeDtypeStruct((4 * NW, EPW), jnp.int32),
    mesh=_mesh,
    compiler_params=pltpu.CompilerParams(needs_layout_passes=False),
    scratch_types=[
        pltpu.VMEM((N * 4,), jnp.float32),
        pltpu.VMEM((EPW,), jnp.int32),
        pltpu.VMEM((EPW,), jnp.int32),
        pltpu.VMEM((4, EPW), jnp.int32),
        pltpu.SemaphoreType.DMA,
    ],
)


def kernel(x, edge_index, W1, b1, gamma, beta, W_fc, b_fc):
    out_dtype = jnp.result_type(x.dtype, W1.dtype, W_fc.dtype)
    W1 = W1.astype(jnp.float32)
    W_fc = W_fc.astype(jnp.float32)
    ei = edge_index.astype(jnp.int32)
    src, dst = ei[0], ei[1]
    pad = NW * EPT - E
    srcp = jnp.concatenate([src, jnp.zeros((pad,), jnp.int32)]).reshape(
        NW, EPT)
    dstp = jnp.concatenate([dst, jnp.full((pad,), DPAD, jnp.int32)]).reshape(
        NW, NG, GRP, SCH)
    src2 = src.reshape(NW, EPW)
    dst2 = dst.reshape(NW, EPW)

    deg2 = _deg_call(_deg_body)(dst2)
    degt = deg2.T

    y = _y_call(x, W1, degt)
    agg = _agg_call(_agg_body)(y, srcp, dstp)

    h, st = _h_call(agg, y, degt, b1.reshape(1, H))
    wcat = jnp.concatenate([W_fc[:H], W_fc[H:]], axis=1)
    bfc = jnp.pad(b_fc, (0, 2)).reshape(1, 4)
    pq = _pq_call(h, st, gamma.reshape(1, H), beta.reshape(1, H), wcat, bfc)

    bits = _score_call(_score_body)(pq.reshape(-1), src2, dst2)
    pairs = jnp.transpose(bits.reshape(2, 2, E), (2, 1, 0))
    out64 = lax.bitcast_convert_type(pairs, jnp.float64)
    return out64 if out_dtype == jnp.float64 else out64.astype(out_dtype)


# speedup vs baseline: 1.2515x; 1.2515x over previous
import functools

import jax
import jax.numpy as jnp
from jax import lax
from jax.experimental import pallas as pl
from jax.experimental.pallas import tpu as pltpu
from jax.experimental.pallas import tpu_sc as plsc

N = 10000
E = 320000
D = 128
H = 128
EPS = 1e-5

NC = 2
NS = 16
NW = NC * NS
EPW = E // NW

_mesh = plsc.VectorSubcoreMesh(
    core_axis_name="c", subcore_axis_name="s", num_cores=NC, num_subcores=NS)


def _deg_body(dst_hbm, deg_hbm, dst_v, hist_v, sem):
    c = lax.axis_index("c")
    s = lax.axis_index("s")
    wid = c * jnp.int32(NS) + s

    zero16 = jnp.zeros((16,), jnp.float32)

    def zb(i, carry):
        hist_v[pl.ds(i * jnp.int32(16), 16)] = zero16
        return carry

    lax.fori_loop(jnp.int32(0), jnp.int32(N // 16), zb, jnp.int32(0))

    pltpu.sync_copy(dst_hbm.at[wid], dst_v)
    ones16 = jnp.ones((16,), jnp.float32)

    def body(i, carry):
        idx = dst_v[pl.ds(i * jnp.int32(16), 16)]
        plsc.addupdate_scatter(hist_v, [idx], ones16)
        return carry

    lax.fori_loop(jnp.int32(0), jnp.int32(EPW // 16), body, jnp.int32(0))

    pltpu.sync_copy(hist_v, deg_hbm.at[wid])


_deg_call = functools.partial(
    pl.kernel,
    out_type=jax.ShapeDtypeStruct((NW, N), jnp.float32),
    mesh=_mesh,
    compiler_params=pltpu.CompilerParams(needs_layout_passes=False),
    scratch_types=[
        pltpu.VMEM((EPW,), jnp.int32),
        pltpu.VMEM((N,), jnp.float32),
        pltpu.SemaphoreType.DMA,
    ],
)


def _y_body(x_ref, w_ref, degt_ref, y_ref):
    deg = jnp.sum(degt_ref[:, :], axis=1, keepdims=True) + 1.0
    dinv = lax.rsqrt(deg)
    xw = jnp.dot(x_ref[:, :], w_ref[:, :], preferred_element_type=jnp.float32,
                 precision=lax.Precision.HIGHEST)
    y_ref[:, :] = xw * dinv


RB = 2000


def _y_call(x, w1, degt):
    grid = N // RB
    return pl.pallas_call(
        _y_body,
        grid=(grid,),
        in_specs=[
            pl.BlockSpec((RB, D), lambda i: (i, jnp.int32(0))),
            pl.BlockSpec((D, H), lambda i: (jnp.int32(0), jnp.int32(0))),
            pl.BlockSpec((RB, NW), lambda i: (i, jnp.int32(0))),
        ],
        out_specs=pl.BlockSpec((RB, H), lambda i: (i, jnp.int32(0))),
        out_shape=jax.ShapeDtypeStruct((N, H), jnp.float32),
    )(x, w1, degt)


EPT = 10112
SCH = 64
NCH2 = EPT // SCH
NB = 2
NP = 10112
RPS = NP // NS
DPAD = 10016


def _agg_body(y_hbm, src_hbm, dst_hbm, agg_hbm, srcb, dstb, rows, shared,
              s0, s1):
    c = lax.axis_index("c")
    s = lax.axis_index("s")
    wid = c * jnp.int32(NS) + s
    sems = (s0, s1)

    zero16 = jnp.zeros((16,), jnp.float32)

    def zb(i, carry):
        for f in range(H // 16):
            rows[jnp.int32(0), i, pl.ds(f * 16, 16)] = zero16
        return carry

    lax.fori_loop(jnp.int32(0), jnp.int32(SCH), zb, jnp.int32(0))
    for k in range(9):
        pltpu.sync_copy(
            rows.at[jnp.int32(0)],
            shared.at[pl.ds(s * jnp.int32(RPS) + jnp.int32(k * SCH), SCH)])
    pltpu.sync_copy(
        rows.at[jnp.int32(0), pl.ds(0, RPS - 9 * SCH)],
        shared.at[pl.ds(s * jnp.int32(RPS) + jnp.int32(9 * SCH),
                        RPS - 9 * SCH)])

    pltpu.sync_copy(src_hbm.at[wid], srcb)
    pltpu.sync_copy(dst_hbm.at[wid], dstb)
    plsc.subcore_barrier()

    for b in range(NB):
        pltpu.async_copy(
            y_hbm.at[srcb.at[pl.ds(jnp.int32(b * SCH), SCH)]],
            rows.at[jnp.int32(b)], sems[b])

    def grp(g, carry):
        for b in range(NB):
            j = g * jnp.int32(NB) + jnp.int32(b)
            pltpu.make_async_copy(
                y_hbm.at[srcb.at[pl.ds(j * jnp.int32(SCH), SCH)]],
                rows.at[jnp.int32(b)], sems[b]).wait()
            pltpu.sync_copy(rows.at[jnp.int32(b)], shared.at[dstb.at[j]],
                            add=True)

            @pl.when(j + jnp.int32(NB) < jnp.int32(NCH2))
            def _():
                pltpu.async_copy(
                    y_hbm.at[srcb.at[pl.ds((j + jnp.int32(NB)) * jnp.int32(SCH),
                                           SCH)]],
                    rows.at[jnp.int32(b)], sems[b])
        return carry

    lax.fori_loop(jnp.int32(0), jnp.int32(NCH2 // NB), grp, jnp.int32(0))
    plsc.subcore_barrier()

    base = s * jnp.int32(RPS)
    pltpu.sync_copy(shared.at[pl.ds(base, RPS)],
                    agg_hbm.at[c, pl.ds(base, RPS)])


_agg_call = functools.partial(
    pl.kernel,
    out_type=jax.ShapeDtypeStruct((NC, NP, H), jnp.float32),
    mesh=_mesh,
    scratch_types=[
        pltpu.VMEM((EPT,), jnp.int32),
        pltpu.VMEM((NCH2, SCH), jnp.int32),
        pltpu.VMEM((NB, SCH, H), jnp.float32),
        pltpu.VMEM_SHARED((NP, H), jnp.float32),
        pltpu.SemaphoreType.DMA,
        pltpu.SemaphoreType.DMA,
    ],
)


def _h_body(agg_ref, y_ref, degt_ref, b1_ref, h_ref, st_ref):
    deg = jnp.sum(degt_ref[:, :], axis=1, keepdims=True) + 1.0
    dinv = lax.rsqrt(deg)
    tot = (agg_ref[0, :, :] + agg_ref[1, :, :] + y_ref[:, :]) * dinv
    h = jnp.maximum(tot + b1_ref[:, :], 0.0)
    h_ref[:, :] = h

    @pl.when(pl.program_id(0) == 0)
    def _():
        st_ref[:, :] = jnp.zeros_like(st_ref)

    st_ref[0:1, :] += jnp.sum(h, axis=0, keepdims=True)
    st_ref[1:2, :] += jnp.sum(h * h, axis=0, keepdims=True)


def _h_call(agg, y, degt, b1r):
    grid = N // RB
    return pl.pallas_call(
        _h_body,
        grid=(grid,),
        in_specs=[
            pl.BlockSpec((NC, RB, H), lambda i: (jnp.int32(0), i, jnp.int32(0))),
            pl.BlockSpec((RB, H), lambda i: (i, jnp.int32(0))),
            pl.BlockSpec((RB, NW), lambda i: (i, jnp.int32(0))),
            pl.BlockSpec((1, H), lambda i: (jnp.int32(0), jnp.int32(0))),
        ],
        out_specs=[
            pl.BlockSpec((RB, H), lambda i: (i, jnp.int32(0))),
            pl.BlockSpec((2, H), lambda i: (jnp.int32(0), jnp.int32(0))),
        ],
        out_shape=[
            jax.ShapeDtypeStruct((N, H), jnp.float32),
            jax.ShapeDtypeStruct((2, H), jnp.float32),
        ],
    )(agg, y, degt, b1r)


def _pq_body(h_ref, st_ref, g_ref, be_ref, w_ref, bfc_ref, pq_ref):
    inv_n = 1.0 / N
    mean = st_ref[0:1, :] * inv_n
    var = st_ref[1:2, :] * inv_n - mean * mean
    scale = g_ref[:, :] * lax.rsqrt(var + EPS)
    shift = be_ref[:, :] - mean * scale
    h2 = jnp.maximum(h_ref[:, :] * scale + shift, 0.0)
    pq_ref[:, :] = (jnp.dot(h2, w_ref[:, :],
                            preferred_element_type=jnp.float32,
                            precision=lax.Precision.HIGHEST)
                    + bfc_ref[:, :])


def _pq_call(h, st, gr, br, wcat, bfc):
    grid = N // RB
    return pl.pallas_call(
        _pq_body,
        grid=(grid,),
        in_specs=[
            pl.BlockSpec((RB, H), lambda i: (i, jnp.int32(0))),
            pl.BlockSpec((2, H), lambda i: (jnp.int32(0), jnp.int32(0))),
            pl.BlockSpec((1, H), lambda i: (jnp.int32(0), jnp.int32(0))),
            pl.BlockSpec((1, H), lambda i: (jnp.int32(0), jnp.int32(0))),
            pl.BlockSpec((H, 4), lambda i: (jnp.int32(0), jnp.int32(0))),
            pl.BlockSpec((1, 4), lambda i: (jnp.int32(0), jnp.int32(0))),
        ],
        out_specs=pl.BlockSpec((RB, 4), lambda i: (i, jnp.int32(0))),
        out_shape=jax.ShapeDtypeStruct((N, 4), jnp.float32),
    )(h, st, gr, br, wcat, bfc)


def _score_body(pq_hbm, src_hbm, dst_hbm, out_hbm, pqv, srcv, dstv, outv, sem):
    c = lax.axis_index("c")
    s = lax.axis_index("s")
    wid = c * jnp.int32(NS) + s

    pltpu.sync_copy(pq_hbm, pqv)
    pltpu.sync_copy(src_hbm.at[wid], srcv)
    pltpu.sync_copy(dst_hbm.at[wid], dstv)

    lane = lax.iota(jnp.int32, 16)
    expmask = jnp.int32(0x7F800000)
    magmask = jnp.int32(0x7FFFFFFF)
    signmask = jnp.int32(-2147483648)
    ebias = jnp.int32(896 << 20)
    zero16i = jnp.zeros((16,), jnp.int32)

    def f64bits(v):
        b = plsc.bitcast(v, jnp.int32)
        nz = (b & expmask) != zero16i
        hi = (b & signmask) | (((b & magmask) >> jnp.int32(3)) + ebias)
        lo = (b & jnp.int32(7)) << jnp.int32(29)
        return jnp.where(nz, lo, zero16i), jnp.where(nz, hi, zero16i)

    def body(i, carry):
        sv = srcv[pl.ds(i * jnp.int32(16), 16)] * jnp.int32(4)
        dv = dstv[pl.ds(i * jnp.int32(16), 16)] * jnp.int32(4)
        p0 = plsc.load_gather(pqv, [sv])
        p1 = plsc.load_gather(pqv, [sv + jnp.int32(1)])
        q0 = plsc.load_gather(pqv, [dv + jnp.int32(2)])
        q1 = plsc.load_gather(pqv, [dv + jnp.int32(3)])
        lo0, hi0 = f64bits(p0 + q0)
        lo1, hi1 = f64bits(p1 + q1)
        base = i * jnp.int32(16)
        outv[0, pl.ds(base, 16)] = lo0
        outv[1, pl.ds(base, 16)] = lo1
        outv[2, pl.ds(base, 16)] = hi0
        outv[3, pl.ds(base, 16)] = hi1
        return carry

    lax.fori_loop(jnp.int32(0), jnp.int32(EPW // 16), body, jnp.int32(0))
    for p in range(4):
        pltpu.sync_copy(outv.at[jnp.int32(p)],
                        out_hbm.at[jnp.int32(p * NW) + wid])


_score_call = functools.partial(
    pl.kernel,
    out_type=jax.ShapeDtypeStruct((4 * NW, EPW), jnp.int32),
    mesh=_mesh,
    compiler_params=pltpu.CompilerParams(needs_layout_passes=False),
    scratch_types=[
        pltpu.VMEM((N * 4,), jnp.float32),
        pltpu.VMEM((EPW,), jnp.int32),
        pltpu.VMEM((EPW,), jnp.int32),
        pltpu.VMEM((4, EPW), jnp.int32),
        pltpu.SemaphoreType.DMA,
    ],
)


def kernel(x, edge_index, W1, b1, gamma, beta, W_fc, b_fc):
    out_dtype = jnp.result_type(x.dtype, W1.dtype, W_fc.dtype)
    W1 = W1.astype(jnp.float32)
    W_fc = W_fc.astype(jnp.float32)
    ei = edge_index.astype(jnp.int32)
    src, dst = ei[0], ei[1]
    pad = NW * EPT - E
    srcp = jnp.concatenate([src, jnp.zeros((pad,), jnp.int32)]).reshape(
        NW, EPT)
    dstp = jnp.concatenate([dst, jnp.full((pad,), DPAD, jnp.int32)]).reshape(
        NW, NCH2, SCH)
    src2 = src.reshape(NW, EPW)
    dst2 = dst.reshape(NW, EPW)

    deg2 = _deg_call(_deg_body)(dst2)
    degt = deg2.T

    y = _y_call(x, W1, degt)
    agg = _agg_call(_agg_body)(y, srcp, dstp)

    h, st = _h_call(agg, y, degt, b1.reshape(1, H))
    wcat = jnp.concatenate([W_fc[:H], W_fc[H:]], axis=1)
    bfc = jnp.pad(b_fc, (0, 2)).reshape(1, 4)
    pq = _pq_call(h, st, gamma.reshape(1, H), beta.reshape(1, H), wcat, bfc)

    bits = _score_call(_score_body)(pq.reshape(-1), src2, dst2)
    pairs = jnp.transpose(bits.reshape(2, 2, E), (2, 1, 0))
    out64 = lax.bitcast_convert_type(pairs, jnp.float64)
    return out64 if out_dtype == jnp.float64 else out64.astype(out_dtype)
